# Initial kernel scaffold; baseline (speedup 1.0000x reference)
import jax
import jax.numpy as jnp
from jax.experimental import pallas as pl


def kernel(x, edge_index, batch, W1, b1, W2, b2, W3, b3, W4, b4):
    raise NotImplementedError("write your pallas kernel here")


# speedup vs baseline: 16.4525x; 16.4525x over previous
import functools

import jax
import jax.numpy as jnp
from jax import lax
from jax.experimental import pallas as pl
from jax.experimental.pallas import tpu as pltpu
from jax.experimental.pallas import tpu_sc as plsc

N = 10000
E = 320000
G = 64

NW = 32
CH = 128
NCH = 80
EP = NW * NCH * CH
NP = 10240
RPT = NP // 16

@functools.cache
def _sc_mesh():
    return plsc.VectorSubcoreMesh(
        core_axis_name="c", subcore_axis_name="s", num_cores=2, num_subcores=16
    )


def _zero_stage(stage_v, f):
    def zloop(i, _):
        for k in range(f // 16):
            stage_v[i, pl.ds(k * 16, 16)] = jnp.zeros((16,), jnp.float32)
        return _

    lax.fori_loop(0, RPT, zloop, None)


@functools.cache
def _make_deg():
    @functools.partial(
        pl.kernel,
        out_type=jax.ShapeDtypeStruct((2, NP, 16), jnp.float32),
        mesh=_sc_mesh(),
        compiler_params=pltpu.CompilerParams(use_tc_tiling_on_sc=False),
        scratch_types=[
            pltpu.VMEM((NCH, CH), jnp.int32),
            pltpu.VMEM((CH, 16), jnp.float32),
            pltpu.VMEM((RPT, 16), jnp.float32),
            pltpu.VMEM_SHARED((NP, 16), jnp.float32),
        ],
    )
    def deg_kernel(dsts_hbm, out_hbm, idx_v, ones_v, stage_v, acc_sh):
        c = lax.axis_index("c")
        s = lax.axis_index("s")
        w = c * 16 + s

        _zero_stage(stage_v, 16)

        def oloop(i, _):
            ones_v[i, :] = jnp.ones((16,), jnp.float32)
            return _

        lax.fori_loop(0, CH, oloop, None)

        pltpu.sync_copy(stage_v, acc_sh.at[pl.ds(s * RPT, RPT)])
        pltpu.sync_copy(dsts_hbm.at[w], idx_v)
        plsc.subcore_barrier()

        def body(j, _):
            pltpu.sync_copy(ones_v, acc_sh.at[idx_v.at[j]], add=True)
            return _

        lax.fori_loop(0, NCH, body, None)
        plsc.subcore_barrier()
        pltpu.sync_copy(acc_sh.at[pl.ds(s * RPT, RPT)],
                        out_hbm.at[c, pl.ds(s * RPT, RPT)])

    return deg_kernel


@functools.cache
def _make_agg(f):
    @functools.partial(
        pl.kernel,
        out_type=jax.ShapeDtypeStruct((2, NP, f), jnp.float32),
        mesh=_sc_mesh(),
        compiler_params=pltpu.CompilerParams(use_tc_tiling_on_sc=False),
        scratch_types=[
            pltpu.VMEM((NCH, CH), jnp.int32),
            pltpu.VMEM((NCH, CH), jnp.int32),
            pltpu.VMEM((CH, f), jnp.float32),
            pltpu.VMEM((RPT, f), jnp.float32),
            pltpu.SemaphoreType.DMA,
            pltpu.VMEM_SHARED((NP, f), jnp.float32),
        ],
    )
    def agg(table_hbm, srcs_hbm, dsts_hbm, out_hbm, src_v, dst_v, rows_v,
            stage_v, sem, acc_sh):
        c = lax.axis_index("c")
        s = lax.axis_index("s")
        w = c * 16 + s

        _zero_stage(stage_v, f)
        pltpu.sync_copy(stage_v, acc_sh.at[pl.ds(s * RPT, RPT)])
        pltpu.sync_copy(srcs_hbm.at[w], src_v)
        pltpu.sync_copy(dsts_hbm.at[w], dst_v)
        plsc.subcore_barrier()

        def body(j, _):
            pltpu.async_copy(table_hbm.at[src_v.at[j]], rows_v, sem).wait()
            pltpu.sync_copy(rows_v, acc_sh.at[dst_v.at[j]], add=True)
            return _

        lax.fori_loop(0, NCH, body, None)
        plsc.subcore_barrier()
        pltpu.sync_copy(acc_sh.at[pl.ds(s * RPT, RPT)],
                        out_hbm.at[c, pl.ds(s * RPT, RPT)])

    return agg


_BLK = 1000
_NBLK = N // _BLK


def _first_tc(x, W1, degp):
    def body(x_ref, w_ref, d0_ref, d1_ref, t_ref, dinv_ref):
        deg = d0_ref[0][:, :1] + d1_ref[0][:, :1] + 1.0
        dinv = lax.rsqrt(deg)
        xw = jnp.dot(x_ref[...], w_ref[...], preferred_element_type=jnp.float32)
        t_ref[...] = xw * dinv
        dinv_ref[...] = dinv

    return pl.pallas_call(
        body,
        grid=(_NBLK,),
        in_specs=[
            pl.BlockSpec((_BLK, 128), lambda i: (i, 0)),
            pl.BlockSpec((128, 16), lambda i: (0, 0)),
            pl.BlockSpec((1, _BLK, 16), lambda i: (0, i, 0)),
            pl.BlockSpec((1, _BLK, 16), lambda i: (1, i, 0)),
        ],
        out_specs=[
            pl.BlockSpec((_BLK, 16), lambda i: (i, 0)),
            pl.BlockSpec((_BLK, 1), lambda i: (i, 0)),
        ],
        out_shape=[
            jax.ShapeDtypeStruct((N, 16), jnp.float32),
            jax.ShapeDtypeStruct((N, 1), jnp.float32),
        ],
    )(x, W1, degp, degp)


def _mid_tc(acc, t, dinv, b, Wn, f_in, f_out):
    def body(a0_ref, a1_ref, t_ref, dinv_ref, b_ref, w_ref, o_ref):
        dinv = dinv_ref[...]
        h = dinv * (a0_ref[0] + a1_ref[0] + t_ref[...]) + b_ref[...]
        h = jnp.maximum(h, 0.0)
        o_ref[...] = jnp.dot(h, w_ref[...], preferred_element_type=jnp.float32) * dinv

    return pl.pallas_call(
        body,
        grid=(_NBLK,),
        in_specs=[
            pl.BlockSpec((1, _BLK, f_in), lambda i: (0, i, 0)),
            pl.BlockSpec((1, _BLK, f_in), lambda i: (1, i, 0)),
            pl.BlockSpec((_BLK, f_in), lambda i: (i, 0)),
            pl.BlockSpec((_BLK, 1), lambda i: (i, 0)),
            pl.BlockSpec((1, f_in), lambda i: (0, 0)),
            pl.BlockSpec((f_in, f_out), lambda i: (0, 0)),
        ],
        out_specs=pl.BlockSpec((_BLK, f_out), lambda i: (i, 0)),
        out_shape=jax.ShapeDtypeStruct((N, f_out), jnp.float32),
    )(acc, acc, t, dinv, b, Wn)


def _final_tc(acc, t, dinv, b, batch2d):
    def body(a0_ref, a1_ref, t_ref, dinv_ref, b_ref, bt_ref, o_ref, pool_ref):
        i = pl.program_id(0)

        @pl.when(i == 0)
        def _():
            pool_ref[...] = jnp.full((G, 16), -jnp.inf, jnp.float32)

        h = dinv_ref[...] * (a0_ref[0] + a1_ref[0] + t_ref[...]) + b_ref[...]
        bt = bt_ref[...]
        for g in range(G):
            cur = jnp.max(jnp.where(bt == g, h, -jnp.inf), axis=0)
            pool_ref[g, :] = jnp.maximum(pool_ref[g, :], cur)

        @pl.when(i == _NBLK - 1)
        def _():
            p = pool_ref[...]
            valid = lax.broadcasted_iota(jnp.int32, (G, 16), 1) < 10
            pm = jnp.where(valid, p, -jnp.inf)
            mx = jnp.max(pm, axis=1, keepdims=True)
            lse = jnp.log(jnp.sum(jnp.where(valid, jnp.exp(pm - mx), 0.0),
                                  axis=1, keepdims=True)) + mx
            o_ref[...] = (p - lse)[:, :10]

    return pl.pallas_call(
        body,
        grid=(_NBLK,),
        in_specs=[
            pl.BlockSpec((1, _BLK, 16), lambda i: (0, i, 0)),
            pl.BlockSpec((1, _BLK, 16), lambda i: (1, i, 0)),
            pl.BlockSpec((_BLK, 16), lambda i: (i, 0)),
            pl.BlockSpec((_BLK, 1), lambda i: (i, 0)),
            pl.BlockSpec((1, 16), lambda i: (0, 0)),
            pl.BlockSpec((_BLK, 1), lambda i: (i, 0)),
        ],
        out_specs=pl.BlockSpec((G, 10), lambda i: (0, 0)),
        out_shape=jax.ShapeDtypeStruct((G, 10), jnp.float32),
        scratch_shapes=[pltpu.VMEM((G, 16), jnp.float32)],
    )(acc, acc, t, dinv, b, batch2d)


def kernel(x, edge_index, batch, W1, b1, W2, b2, W3, b3, W4, b4):
    src = edge_index[0]
    dst = edge_index[1]
    pad = EP - E
    srcs = jnp.concatenate([src, jnp.zeros((pad,), jnp.int32)]).reshape(NW, NCH, CH)
    dsts = jnp.concatenate([dst, jnp.full((pad,), N, jnp.int32)]).reshape(NW, NCH, CH)

    degp = _make_deg()(dsts)

    t1, dinv = _first_tc(x, W1, degp)
    a1 = _make_agg(16)(t1, srcs, dsts)
    t2 = _mid_tc(a1, t1, dinv, b1.reshape(1, 16), W2, 16, 32)
    a2 = _make_agg(32)(t2, srcs, dsts)
    t3 = _mid_tc(a2, t2, dinv, b2.reshape(1, 32), W3, 32, 64)
    a3 = _make_agg(64)(t3, srcs, dsts)
    W4p = jnp.pad(W4, ((0, 0), (0, 6)))
    b4p = jnp.pad(b4, (0, 6))
    t4 = _mid_tc(a3, t3, dinv, b3.reshape(1, 64), W4p, 64, 16)
    a4 = _make_agg(16)(t4, srcs, dsts)
    return _final_tc(a4, t4, dinv, b4p.reshape(1, 16), batch.reshape(N, 1))


# speedup vs baseline: 18.7594x; 1.1402x over previous
import functools

import jax
import jax.numpy as jnp
from jax import lax
from jax.experimental import pallas as pl
from jax.experimental.pallas import tpu as pltpu
from jax.experimental.pallas import tpu_sc as plsc

N = 10000
E = 320000
G = 64

NW = 32
CH = 128
NCH = 80
EP = NW * NCH * CH
NP = 10240
RPT = NP // 16

@functools.cache
def _sc_mesh():
    return plsc.VectorSubcoreMesh(
        core_axis_name="c", subcore_axis_name="s", num_cores=2, num_subcores=16
    )


def _zero_stage(stage_v, f):
    def zloop(i, _):
        for k in range(f // 16):
            stage_v[i, pl.ds(k * 16, 16)] = jnp.zeros((16,), jnp.float32)
        return _

    lax.fori_loop(0, RPT, zloop, None)


@functools.cache
def _make_deg():
    @functools.partial(
        pl.kernel,
        out_type=jax.ShapeDtypeStruct((2, NP, 16), jnp.float32),
        mesh=_sc_mesh(),
        compiler_params=pltpu.CompilerParams(use_tc_tiling_on_sc=False),
        scratch_types=[
            pltpu.VMEM((NCH, CH), jnp.int32),
            pltpu.VMEM((CH, 16), jnp.float32),
            pltpu.VMEM((RPT, 16), jnp.float32),
            pltpu.VMEM_SHARED((NP, 16), jnp.float32),
        ],
    )
    def deg_kernel(dsts_hbm, out_hbm, idx_v, ones_v, stage_v, acc_sh):
        c = lax.axis_index("c")
        s = lax.axis_index("s")
        w = c * 16 + s

        _zero_stage(stage_v, 16)

        def oloop(i, _):
            ones_v[i, :] = jnp.ones((16,), jnp.float32)
            return _

        lax.fori_loop(0, CH, oloop, None)

        pltpu.sync_copy(stage_v, acc_sh.at[pl.ds(s * RPT, RPT)])
        pltpu.sync_copy(dsts_hbm.at[w], idx_v)
        plsc.subcore_barrier()

        def body(j, _):
            pltpu.sync_copy(ones_v, acc_sh.at[idx_v.at[j]], add=True)
            return _

        lax.fori_loop(0, NCH, body, None)
        plsc.subcore_barrier()
        pltpu.sync_copy(acc_sh.at[pl.ds(s * RPT, RPT)],
                        out_hbm.at[c, pl.ds(s * RPT, RPT)])

    return deg_kernel


@functools.cache
def _make_agg(f):
    nbuf = 4

    @functools.partial(
        pl.kernel,
        out_type=jax.ShapeDtypeStruct((2, NP, f), jnp.float32),
        mesh=_sc_mesh(),
        compiler_params=pltpu.CompilerParams(use_tc_tiling_on_sc=False),
        scratch_types=[
            pltpu.VMEM((NCH, CH), jnp.int32),
            pltpu.VMEM((NCH, CH), jnp.int32),
            [pltpu.VMEM((CH, f), jnp.float32) for _ in range(nbuf)],
            [pltpu.SemaphoreType.DMA for _ in range(nbuf)],
            pltpu.VMEM_SHARED((NP, f), jnp.float32),
        ],
    )
    def agg(table_hbm, srcs_hbm, dsts_hbm, out_hbm, src_v, dst_v, rows,
            sems, acc_sh):
        c = lax.axis_index("c")
        s = lax.axis_index("s")
        w = c * 16 + s

        def zloop(i, _):
            for k in range(f // 16):
                rows[0][i, pl.ds(k * 16, 16)] = jnp.zeros((16,), jnp.float32)
            return _

        lax.fori_loop(0, CH, zloop, None)
        for r in range(RPT // CH):
            pltpu.sync_copy(rows[0], acc_sh.at[pl.ds(s * RPT + r * CH, CH)])
        pltpu.sync_copy(srcs_hbm.at[w], src_v)
        pltpu.sync_copy(dsts_hbm.at[w], dst_v)
        plsc.subcore_barrier()

        def body(jj, _):
            j = jj * nbuf
            cps = [
                pltpu.async_copy(table_hbm.at[src_v.at[j + b]], rows[b], sems[b])
                for b in range(nbuf)
            ]
            for b in range(nbuf):
                cps[b].wait()
                pltpu.sync_copy(rows[b], acc_sh.at[dst_v.at[j + b]], add=True)
            return _

        lax.fori_loop(0, NCH // nbuf, body, None)
        plsc.subcore_barrier()
        pltpu.sync_copy(acc_sh.at[pl.ds(s * RPT, RPT)],
                        out_hbm.at[c, pl.ds(s * RPT, RPT)])

    return agg


_BLK = 1000
_NBLK = N // _BLK


def _first_tc(x, W1, degp):
    def body(x_ref, w_ref, d0_ref, d1_ref, t_ref, dinv_ref):
        deg = d0_ref[0][:, :1] + d1_ref[0][:, :1] + 1.0
        dinv = lax.rsqrt(deg)
        xw = jnp.dot(x_ref[...], w_ref[...], preferred_element_type=jnp.float32)
        t_ref[...] = xw * dinv
        dinv_ref[...] = dinv

    return pl.pallas_call(
        body,
        grid=(_NBLK,),
        in_specs=[
            pl.BlockSpec((_BLK, 128), lambda i: (i, 0)),
            pl.BlockSpec((128, 16), lambda i: (0, 0)),
            pl.BlockSpec((1, _BLK, 16), lambda i: (0, i, 0)),
            pl.BlockSpec((1, _BLK, 16), lambda i: (1, i, 0)),
        ],
        out_specs=[
            pl.BlockSpec((_BLK, 16), lambda i: (i, 0)),
            pl.BlockSpec((_BLK, 1), lambda i: (i, 0)),
        ],
        out_shape=[
            jax.ShapeDtypeStruct((N, 16), jnp.float32),
            jax.ShapeDtypeStruct((N, 1), jnp.float32),
        ],
    )(x, W1, degp, degp)


def _mid_tc(acc, t, dinv, b, Wn, f_in, f_out):
    def body(a0_ref, a1_ref, t_ref, dinv_ref, b_ref, w_ref, o_ref):
        dinv = dinv_ref[...]
        h = dinv * (a0_ref[0] + a1_ref[0] + t_ref[...]) + b_ref[...]
        h = jnp.maximum(h, 0.0)
        o_ref[...] = jnp.dot(h, w_ref[...], preferred_element_type=jnp.float32) * dinv

    return pl.pallas_call(
        body,
        grid=(_NBLK,),
        in_specs=[
            pl.BlockSpec((1, _BLK, f_in), lambda i: (0, i, 0)),
            pl.BlockSpec((1, _BLK, f_in), lambda i: (1, i, 0)),
            pl.BlockSpec((_BLK, f_in), lambda i: (i, 0)),
            pl.BlockSpec((_BLK, 1), lambda i: (i, 0)),
            pl.BlockSpec((1, f_in), lambda i: (0, 0)),
            pl.BlockSpec((f_in, f_out), lambda i: (0, 0)),
        ],
        out_specs=pl.BlockSpec((_BLK, f_out), lambda i: (i, 0)),
        out_shape=jax.ShapeDtypeStruct((N, f_out), jnp.float32),
    )(acc, acc, t, dinv, b, Wn)


def _final_tc(acc, t, dinv, b, batch2d):
    def body(a0_ref, a1_ref, t_ref, dinv_ref, b_ref, bt_ref, o_ref, pool_ref):
        i = pl.program_id(0)

        @pl.when(i == 0)
        def _():
            pool_ref[...] = jnp.full((G, 16), -jnp.inf, jnp.float32)

        h = dinv_ref[...] * (a0_ref[0] + a1_ref[0] + t_ref[...]) + b_ref[...]
        bt = bt_ref[...]
        for g in range(G):
            cur = jnp.max(jnp.where(bt == g, h, -jnp.inf), axis=0)
            pool_ref[g, :] = jnp.maximum(pool_ref[g, :], cur)

        @pl.when(i == _NBLK - 1)
        def _():
            p = pool_ref[...]
            valid = lax.broadcasted_iota(jnp.int32, (G, 16), 1) < 10
            pm = jnp.where(valid, p, -jnp.inf)
            mx = jnp.max(pm, axis=1, keepdims=True)
            lse = jnp.log(jnp.sum(jnp.where(valid, jnp.exp(pm - mx), 0.0),
                                  axis=1, keepdims=True)) + mx
            o_ref[...] = (p - lse)[:, :10]

    return pl.pallas_call(
        body,
        grid=(_NBLK,),
        in_specs=[
            pl.BlockSpec((1, _BLK, 16), lambda i: (0, i, 0)),
            pl.BlockSpec((1, _BLK, 16), lambda i: (1, i, 0)),
            pl.BlockSpec((_BLK, 16), lambda i: (i, 0)),
            pl.BlockSpec((_BLK, 1), lambda i: (i, 0)),
            pl.BlockSpec((1, 16), lambda i: (0, 0)),
            pl.BlockSpec((_BLK, 1), lambda i: (i, 0)),
        ],
        out_specs=pl.BlockSpec((G, 10), lambda i: (0, 0)),
        out_shape=jax.ShapeDtypeStruct((G, 10), jnp.float32),
        scratch_shapes=[pltpu.VMEM((G, 16), jnp.float32)],
    )(acc, acc, t, dinv, b, batch2d)


def kernel(x, edge_index, batch, W1, b1, W2, b2, W3, b3, W4, b4):
    src = edge_index[0]
    dst = edge_index[1]
    pad = EP - E
    srcs = jnp.concatenate([src, jnp.zeros((pad,), jnp.int32)]).reshape(NW, NCH, CH)
    dsts = jnp.concatenate([dst, jnp.full((pad,), N, jnp.int32)]).reshape(NW, NCH, CH)

    degp = _make_deg()(dsts)

    t1, dinv = _first_tc(x, W1, degp)
    a1 = _make_agg(16)(t1, srcs, dsts)
    t2 = _mid_tc(a1, t1, dinv, b1.reshape(1, 16), W2, 16, 32)
    a2 = _make_agg(32)(t2, srcs, dsts)
    t3 = _mid_tc(a2, t2, dinv, b2.reshape(1, 32), W3, 32, 64)
    a3 = _make_agg(64)(t3, srcs, dsts)
    W4p = jnp.pad(W4, ((0, 0), (0, 6)))
    b4p = jnp.pad(b4, (0, 6))
    t4 = _mid_tc(a3, t3, dinv, b3.reshape(1, 64), W4p, 64, 16)
    a4 = _make_agg(16)(t4, srcs, dsts)
    return _final_tc(a4, t4, dinv, b4p.reshape(1, 16), batch.reshape(N, 1))


# speedup vs baseline: 21.4546x; 1.1437x over previous
import functools

import jax
import jax.numpy as jnp
from jax import lax
from jax.experimental import pallas as pl
from jax.experimental.pallas import tpu as pltpu
from jax.experimental.pallas import tpu_sc as plsc

N = 10000
E = 320000
G = 64

NW = 32
CH = 128
NCH = 80
EP = NW * NCH * CH
NP = 10240
RPT = NP // 16

@functools.cache
def _sc_mesh():
    return plsc.VectorSubcoreMesh(
        core_axis_name="c", subcore_axis_name="s", num_cores=2, num_subcores=16
    )


def _zero_stage(stage_v, f):
    def zloop(i, _):
        for k in range(f // 16):
            stage_v[i, pl.ds(k * 16, 16)] = jnp.zeros((16,), jnp.float32)
        return _

    lax.fori_loop(0, RPT, zloop, None)


def _unpack_edges(pk_v, src_v, dst_v):
    def uloop(i, _):
        for k in range(CH // 16):
            v = pk_v[i, pl.ds(k * 16, 16)]
            src_v[i, pl.ds(k * 16, 16)] = v & 0xFFFF
            dst_v[i, pl.ds(k * 16, 16)] = v >> 16
        return _

    lax.fori_loop(0, NCH, uloop, None)


@functools.cache
def _make_deg():
    @functools.partial(
        pl.kernel,
        out_type=jax.ShapeDtypeStruct((2, NP, 16), jnp.float32),
        mesh=_sc_mesh(),
        compiler_params=pltpu.CompilerParams(use_tc_tiling_on_sc=False),
        scratch_types=[
            pltpu.VMEM((NCH, CH), jnp.int32),
            pltpu.VMEM((NCH, CH), jnp.int32),
            pltpu.VMEM((CH, 16), jnp.float32),
            pltpu.VMEM((RPT, 16), jnp.float32),
            pltpu.VMEM_SHARED((NP, 16), jnp.float32),
        ],
    )
    def deg_kernel(pk_hbm, out_hbm, pk_v, idx_v, ones_v, stage_v, acc_sh):
        c = lax.axis_index("c")
        s = lax.axis_index("s")
        w = c * 16 + s

        _zero_stage(stage_v, 16)

        def oloop(i, _):
            ones_v[i, :] = jnp.ones((16,), jnp.float32)
            return _

        lax.fori_loop(0, CH, oloop, None)

        pltpu.sync_copy(stage_v, acc_sh.at[pl.ds(s * RPT, RPT)])
        pltpu.sync_copy(pk_hbm.at[w], pk_v)
        _unpack_edges(pk_v, idx_v, idx_v)
        plsc.subcore_barrier()

        def body(j, _):
            pltpu.sync_copy(ones_v, acc_sh.at[idx_v.at[j]], add=True)
            return _

        lax.fori_loop(0, NCH, body, None)
        plsc.subcore_barrier()
        pltpu.sync_copy(acc_sh.at[pl.ds(s * RPT, RPT)],
                        out_hbm.at[c, pl.ds(s * RPT, RPT)])

    return deg_kernel


@functools.cache
def _make_agg(f, in_spmem):
    nbuf = 4

    @functools.partial(
        pl.kernel,
        out_type=jax.ShapeDtypeStruct((2, NP, f), jnp.float32),
        mesh=_sc_mesh(),
        compiler_params=pltpu.CompilerParams(use_tc_tiling_on_sc=False),
        scratch_types=[
            pltpu.VMEM((NCH, CH), jnp.int32),
            pltpu.VMEM((NCH, CH), jnp.int32),
            pltpu.VMEM((NCH, CH), jnp.int32),
            [pltpu.VMEM((CH, f), jnp.float32) for _ in range(nbuf)],
            [pltpu.SemaphoreType.DMA for _ in range(nbuf)],
            pltpu.VMEM_SHARED((NP, f), jnp.float32),
            (pltpu.VMEM_SHARED((N, f), jnp.float32) if in_spmem
             else pltpu.VMEM((16,), jnp.float32)),
        ],
    )
    def agg(table_hbm, pk_hbm, out_hbm, pk_v, src_v, dst_v, rows,
            sems, acc_sh, table_sh):
        c = lax.axis_index("c")
        s = lax.axis_index("s")
        w = c * 16 + s

        if in_spmem:
            tpt = N // 16
            pltpu.sync_copy(table_hbm.at[pl.ds(s * tpt, tpt)],
                            table_sh.at[pl.ds(s * tpt, tpt)])
        table = table_sh if in_spmem else table_hbm

        def zloop(i, _):
            for k in range(f // 16):
                rows[0][i, pl.ds(k * 16, 16)] = jnp.zeros((16,), jnp.float32)
            return _

        lax.fori_loop(0, CH, zloop, None)
        for r in range(RPT // CH):
            pltpu.sync_copy(rows[0], acc_sh.at[pl.ds(s * RPT + r * CH, CH)])
        pltpu.sync_copy(pk_hbm.at[w], pk_v)
        _unpack_edges(pk_v, src_v, dst_v)
        plsc.subcore_barrier()

        def body(jj, _):
            j = jj * nbuf
            cps = [
                pltpu.async_copy(table.at[src_v.at[j + b]], rows[b], sems[b])
                for b in range(nbuf)
            ]
            for b in range(nbuf):
                cps[b].wait()
                pltpu.sync_copy(rows[b], acc_sh.at[dst_v.at[j + b]], add=True)
            return _

        lax.fori_loop(0, NCH // nbuf, body, None)
        plsc.subcore_barrier()
        pltpu.sync_copy(acc_sh.at[pl.ds(s * RPT, RPT)],
                        out_hbm.at[c, pl.ds(s * RPT, RPT)])

    return agg


_BLK = 1000
_NBLK = N // _BLK


def _first_tc(x, W1, degp):
    def body(x_ref, w_ref, d0_ref, d1_ref, t_ref, dinv_ref):
        deg = d0_ref[0][:, :1] + d1_ref[0][:, :1] + 1.0
        dinv = lax.rsqrt(deg)
        xw = jnp.dot(x_ref[...], w_ref[...], preferred_element_type=jnp.float32)
        t_ref[...] = xw * dinv
        dinv_ref[...] = dinv

    return pl.pallas_call(
        body,
        grid=(_NBLK,),
        in_specs=[
            pl.BlockSpec((_BLK, 128), lambda i: (i, 0)),
            pl.BlockSpec((128, 16), lambda i: (0, 0)),
            pl.BlockSpec((1, _BLK, 16), lambda i: (0, i, 0)),
            pl.BlockSpec((1, _BLK, 16), lambda i: (1, i, 0)),
        ],
        out_specs=[
            pl.BlockSpec((_BLK, 16), lambda i: (i, 0)),
            pl.BlockSpec((_BLK, 1), lambda i: (i, 0)),
        ],
        out_shape=[
            jax.ShapeDtypeStruct((N, 16), jnp.float32),
            jax.ShapeDtypeStruct((N, 1), jnp.float32),
        ],
    )(x, W1, degp, degp)


def _mid_tc(acc, t, dinv, b, Wn, f_in, f_out):
    def body(a0_ref, a1_ref, t_ref, dinv_ref, b_ref, w_ref, o_ref):
        dinv = dinv_ref[...]
        h = dinv * (a0_ref[0] + a1_ref[0] + t_ref[...]) + b_ref[...]
        h = jnp.maximum(h, 0.0)
        o_ref[...] = jnp.dot(h, w_ref[...], preferred_element_type=jnp.float32) * dinv

    return pl.pallas_call(
        body,
        grid=(_NBLK,),
        in_specs=[
            pl.BlockSpec((1, _BLK, f_in), lambda i: (0, i, 0)),
            pl.BlockSpec((1, _BLK, f_in), lambda i: (1, i, 0)),
            pl.BlockSpec((_BLK, f_in), lambda i: (i, 0)),
            pl.BlockSpec((_BLK, 1), lambda i: (i, 0)),
            pl.BlockSpec((1, f_in), lambda i: (0, 0)),
            pl.BlockSpec((f_in, f_out), lambda i: (0, 0)),
        ],
        out_specs=pl.BlockSpec((_BLK, f_out), lambda i: (i, 0)),
        out_shape=jax.ShapeDtypeStruct((N, f_out), jnp.float32),
    )(acc, acc, t, dinv, b, Wn)


def _final_tc(acc, t, dinv, b, batch2d):
    def body(a0_ref, a1_ref, t_ref, dinv_ref, b_ref, bt_ref, o_ref, pool_ref):
        i = pl.program_id(0)

        @pl.when(i == 0)
        def _():
            pool_ref[...] = jnp.full((G, 16), -jnp.inf, jnp.float32)

        h = dinv_ref[...] * (a0_ref[0] + a1_ref[0] + t_ref[...]) + b_ref[...]
        bt = bt_ref[...]
        for g in range(G):
            cur = jnp.max(jnp.where(bt == g, h, -jnp.inf), axis=0)
            pool_ref[g, :] = jnp.maximum(pool_ref[g, :], cur)

        @pl.when(i == _NBLK - 1)
        def _():
            p = pool_ref[...]
            valid = lax.broadcasted_iota(jnp.int32, (G, 16), 1) < 10
            pm = jnp.where(valid, p, -jnp.inf)
            mx = jnp.max(pm, axis=1, keepdims=True)
            lse = jnp.log(jnp.sum(jnp.where(valid, jnp.exp(pm - mx), 0.0),
                                  axis=1, keepdims=True)) + mx
            o_ref[...] = (p - lse)[:, :10]

    return pl.pallas_call(
        body,
        grid=(_NBLK,),
        in_specs=[
            pl.BlockSpec((1, _BLK, 16), lambda i: (0, i, 0)),
            pl.BlockSpec((1, _BLK, 16), lambda i: (1, i, 0)),
            pl.BlockSpec((_BLK, 16), lambda i: (i, 0)),
            pl.BlockSpec((_BLK, 1), lambda i: (i, 0)),
            pl.BlockSpec((1, 16), lambda i: (0, 0)),
            pl.BlockSpec((_BLK, 1), lambda i: (i, 0)),
        ],
        out_specs=pl.BlockSpec((G, 10), lambda i: (0, 0)),
        out_shape=jax.ShapeDtypeStruct((G, 10), jnp.float32),
        scratch_shapes=[pltpu.VMEM((G, 16), jnp.float32)],
    )(acc, acc, t, dinv, b, batch2d)


def kernel(x, edge_index, batch, W1, b1, W2, b2, W3, b3, W4, b4):
    src = edge_index[0]
    dst = edge_index[1]
    pad = EP - E
    packed = jnp.concatenate(
        [(dst << 16) | src, jnp.full((pad,), N << 16, jnp.int32)]
    ).reshape(NW, NCH, CH)

    degp = _make_deg()(packed)

    t1, dinv = _first_tc(x, W1, degp)
    a1 = _make_agg(16, True)(t1, packed)
    t2 = _mid_tc(a1, t1, dinv, b1.reshape(1, 16), W2, 16, 32)
    a2 = _make_agg(32, True)(t2, packed)
    t3 = _mid_tc(a2, t2, dinv, b2.reshape(1, 32), W3, 32, 64)
    a3 = _make_agg(64, False)(t3, packed)
    W4p = jnp.pad(W4, ((0, 0), (0, 6)))
    b4p = jnp.pad(b4, (0, 6))
    t4 = _mid_tc(a3, t3, dinv, b3.reshape(1, 64), W4p, 64, 16)
    a4 = _make_agg(16, True)(t4, packed)
    return _final_tc(a4, t4, dinv, b4p.reshape(1, 16), batch.reshape(N, 1))


# speedup vs baseline: 30.2835x; 1.4115x over previous
import functools

import jax
import jax.numpy as jnp
from jax import lax
from jax.experimental import pallas as pl
from jax.experimental.pallas import tpu as pltpu
from jax.experimental.pallas import tpu_sc as plsc

N = 10000
E = 320000
G = 64

NW = 32
CH = 128
NCH = 80
EP = NW * NCH * CH
NP = 10240
RPT = NP // 16

@functools.cache
def _sc_mesh():
    return plsc.VectorSubcoreMesh(
        core_axis_name="c", subcore_axis_name="s", num_cores=2, num_subcores=16
    )


def _zero_stage(stage_v, f):
    def zloop(i, _):
        for k in range(f // 16):
            stage_v[i, pl.ds(k * 16, 16)] = jnp.zeros((16,), jnp.float32)
        return _

    lax.fori_loop(0, RPT, zloop, None)


def _unpack_edges(pk_v, src_v, dst_v):
    def uloop(i, _):
        for k in range(CH // 16):
            v = pk_v[i, pl.ds(k * 16, 16)]
            src_v[i, pl.ds(k * 16, 16)] = v & 0xFFFF
            dst_v[i, pl.ds(k * 16, 16)] = v >> 16
        return _

    lax.fori_loop(0, NCH, uloop, None)


@functools.cache
def _make_deg():
    @functools.partial(
        pl.kernel,
        out_type=jax.ShapeDtypeStruct((2, NP, 16), jnp.float32),
        mesh=_sc_mesh(),
        compiler_params=pltpu.CompilerParams(use_tc_tiling_on_sc=False),
        scratch_types=[
            pltpu.VMEM((NCH, CH), jnp.int32),
            pltpu.VMEM((NCH, CH), jnp.int32),
            pltpu.VMEM((CH, 16), jnp.float32),
            pltpu.VMEM((RPT, 16), jnp.float32),
            pltpu.VMEM_SHARED((NP, 16), jnp.float32),
        ],
    )
    def deg_kernel(pk_hbm, out_hbm, pk_v, idx_v, ones_v, stage_v, acc_sh):
        c = lax.axis_index("c")
        s = lax.axis_index("s")
        w = c * 16 + s

        _zero_stage(stage_v, 16)

        def oloop(i, _):
            ones_v[i, :] = jnp.ones((16,), jnp.float32)
            return _

        lax.fori_loop(0, CH, oloop, None)

        pltpu.sync_copy(stage_v, acc_sh.at[pl.ds(s * RPT, RPT)])
        pltpu.sync_copy(pk_hbm.at[w], pk_v)
        _unpack_edges(pk_v, idx_v, idx_v)
        plsc.subcore_barrier()

        def body(j, _):
            pltpu.sync_copy(ones_v, acc_sh.at[idx_v.at[j]], add=True)
            return _

        lax.fori_loop(0, NCH, body, None)
        plsc.subcore_barrier()
        pltpu.sync_copy(acc_sh.at[pl.ds(s * RPT, RPT)],
                        out_hbm.at[c, pl.ds(s * RPT, RPT)])

    return deg_kernel


@functools.cache
def _make_agg(f, in_spmem):
    nbuf = 4

    @functools.partial(
        pl.kernel,
        out_type=jax.ShapeDtypeStruct((2, NP, f), jnp.float32),
        mesh=_sc_mesh(),
        compiler_params=pltpu.CompilerParams(use_tc_tiling_on_sc=False),
        scratch_types=[
            pltpu.VMEM((NCH, CH), jnp.int32),
            pltpu.VMEM((NCH, CH), jnp.int32),
            pltpu.VMEM((NCH, CH), jnp.int32),
            [pltpu.VMEM((CH, f), jnp.float32) for _ in range(nbuf)],
            [pltpu.SemaphoreType.DMA for _ in range(nbuf)],
            pltpu.VMEM_SHARED((NP, f), jnp.float32),
            (pltpu.VMEM_SHARED((N, f), jnp.float32) if in_spmem
             else pltpu.VMEM((16,), jnp.float32)),
        ],
    )
    def agg(table_hbm, pk_hbm, out_hbm, pk_v, src_v, dst_v, rows,
            sems, acc_sh, table_sh):
        c = lax.axis_index("c")
        s = lax.axis_index("s")
        w = c * 16 + s

        if in_spmem:
            tpt = N // 16
            pltpu.sync_copy(table_hbm.at[pl.ds(s * tpt, tpt)],
                            table_sh.at[pl.ds(s * tpt, tpt)])
        table = table_sh if in_spmem else table_hbm

        def zloop(i, _):
            for k in range(f // 16):
                rows[0][i, pl.ds(k * 16, 16)] = jnp.zeros((16,), jnp.float32)
            return _

        lax.fori_loop(0, CH, zloop, None)
        for r in range(RPT // CH):
            pltpu.sync_copy(rows[0], acc_sh.at[pl.ds(s * RPT + r * CH, CH)])
        pltpu.sync_copy(pk_hbm.at[w], pk_v)
        _unpack_edges(pk_v, src_v, dst_v)
        plsc.subcore_barrier()

        def body(jj, _):
            j = jj * nbuf
            cps = [
                pltpu.async_copy(table.at[src_v.at[j + b]], rows[b], sems[b])
                for b in range(nbuf)
            ]
            for b in range(nbuf):
                cps[b].wait()
                pltpu.sync_copy(rows[b], acc_sh.at[dst_v.at[j + b]], add=True)
            return _

        lax.fori_loop(0, NCH // nbuf, body, None)
        plsc.subcore_barrier()
        pltpu.sync_copy(acc_sh.at[pl.ds(s * RPT, RPT)],
                        out_hbm.at[c, pl.ds(s * RPT, RPT)])

    return agg


_BLK = 1000
_NBLK = N // _BLK


def _first_tc(x, W1, degp):
    def body(x_ref, w_ref, d0_ref, d1_ref, t_ref, dinv_ref):
        deg = d0_ref[0][:, :1] + d1_ref[0][:, :1] + 1.0
        dinv = lax.rsqrt(deg)
        xw = jnp.dot(x_ref[...], w_ref[...], preferred_element_type=jnp.float32)
        t_ref[...] = xw * dinv
        dinv_ref[...] = dinv

    return pl.pallas_call(
        body,
        grid=(_NBLK,),
        in_specs=[
            pl.BlockSpec((_BLK, 128), lambda i: (i, 0)),
            pl.BlockSpec((128, 16), lambda i: (0, 0)),
            pl.BlockSpec((1, _BLK, 16), lambda i: (0, i, 0)),
            pl.BlockSpec((1, _BLK, 16), lambda i: (1, i, 0)),
        ],
        out_specs=[
            pl.BlockSpec((_BLK, 16), lambda i: (i, 0)),
            pl.BlockSpec((_BLK, 1), lambda i: (i, 0)),
        ],
        out_shape=[
            jax.ShapeDtypeStruct((N, 16), jnp.float32),
            jax.ShapeDtypeStruct((N, 1), jnp.float32),
        ],
    )(x, W1, degp, degp)


def _mid_tc(accs, ts, dinv, b, Wn, f_in, f_out, nout):
    nin = len(ts)
    fh = f_in // nin
    fo = f_out // nout

    def body(*refs):
        a_refs = refs[: 2 * nin]
        t_refs = refs[2 * nin: 3 * nin]
        dinv_ref, b_ref, w_ref = refs[3 * nin: 3 * nin + 3]
        o_refs = refs[3 * nin + 3:]
        dinv = dinv_ref[...]
        parts = []
        for k in range(nin):
            hk = (dinv * (a_refs[2 * k][0] + a_refs[2 * k + 1][0] + t_refs[k][...])
                  + b_ref[:, k * fh:(k + 1) * fh])
            parts.append(hk)
        h = jnp.maximum(parts[0] if nin == 1 else jnp.concatenate(parts, axis=1), 0.0)
        for m in range(nout):
            o_refs[m][...] = jnp.dot(
                h, w_ref[:, m * fo:(m + 1) * fo],
                preferred_element_type=jnp.float32) * dinv

    in_specs = []
    args = []
    for k in range(nin):
        in_specs += [
            pl.BlockSpec((1, _BLK, fh), lambda i: (0, i, 0)),
            pl.BlockSpec((1, _BLK, fh), lambda i: (1, i, 0)),
        ]
        args += [accs[k], accs[k]]
    for k in range(nin):
        in_specs.append(pl.BlockSpec((_BLK, fh), lambda i: (i, 0)))
        args.append(ts[k])
    in_specs += [
        pl.BlockSpec((_BLK, 1), lambda i: (i, 0)),
        pl.BlockSpec((1, f_in), lambda i: (0, 0)),
        pl.BlockSpec((f_in, f_out), lambda i: (0, 0)),
    ]
    args += [dinv, b, Wn]
    out = pl.pallas_call(
        body,
        grid=(_NBLK,),
        in_specs=in_specs,
        out_specs=[pl.BlockSpec((_BLK, fo), lambda i: (i, 0))] * nout,
        out_shape=[jax.ShapeDtypeStruct((N, fo), jnp.float32)] * nout,
    )(*args)
    return out


def _final_tc(acc, t, dinv, b, batch2d):
    def body(a0_ref, a1_ref, t_ref, dinv_ref, b_ref, bt_ref, o_ref, pool_ref):
        i = pl.program_id(0)

        @pl.when(i == 0)
        def _():
            pool_ref[...] = jnp.full((G, 16), -jnp.inf, jnp.float32)

        h = dinv_ref[...] * (a0_ref[0] + a1_ref[0] + t_ref[...]) + b_ref[...]
        bt = bt_ref[...]
        for g in range(G):
            cur = jnp.max(jnp.where(bt == g, h, -jnp.inf), axis=0)
            pool_ref[g, :] = jnp.maximum(pool_ref[g, :], cur)

        @pl.when(i == _NBLK - 1)
        def _():
            p = pool_ref[...]
            valid = lax.broadcasted_iota(jnp.int32, (G, 16), 1) < 10
            pm = jnp.where(valid, p, -jnp.inf)
            mx = jnp.max(pm, axis=1, keepdims=True)
            lse = jnp.log(jnp.sum(jnp.where(valid, jnp.exp(pm - mx), 0.0),
                                  axis=1, keepdims=True)) + mx
            o_ref[...] = (p - lse)[:, :10]

    return pl.pallas_call(
        body,
        grid=(_NBLK,),
        in_specs=[
            pl.BlockSpec((1, _BLK, 16), lambda i: (0, i, 0)),
            pl.BlockSpec((1, _BLK, 16), lambda i: (1, i, 0)),
            pl.BlockSpec((_BLK, 16), lambda i: (i, 0)),
            pl.BlockSpec((_BLK, 1), lambda i: (i, 0)),
            pl.BlockSpec((1, 16), lambda i: (0, 0)),
            pl.BlockSpec((_BLK, 1), lambda i: (i, 0)),
        ],
        out_specs=pl.BlockSpec((G, 10), lambda i: (0, 0)),
        out_shape=jax.ShapeDtypeStruct((G, 10), jnp.float32),
        scratch_shapes=[pltpu.VMEM((G, 16), jnp.float32)],
    )(acc, acc, t, dinv, b, batch2d)


def kernel(x, edge_index, batch, W1, b1, W2, b2, W3, b3, W4, b4):
    src = edge_index[0]
    dst = edge_index[1]
    pad = EP - E
    packed = jnp.concatenate(
        [(dst << 16) | src, jnp.full((pad,), N << 16, jnp.int32)]
    ).reshape(NW, NCH, CH)

    degp = _make_deg()(packed)

    t1, dinv = _first_tc(x, W1, degp)
    a1 = _make_agg(16, True)(t1, packed)
    (t2,) = _mid_tc([a1], [t1], dinv, b1.reshape(1, 16), W2, 16, 32, 1)
    a2 = _make_agg(32, True)(t2, packed)
    t3a, t3b = _mid_tc([a2], [t2], dinv, b2.reshape(1, 32), W3, 32, 64, 2)
    a3a = _make_agg(32, True)(t3a, packed)
    a3b = _make_agg(32, True)(t3b, packed)
    W4p = jnp.pad(W4, ((0, 0), (0, 6)))
    b4p = jnp.pad(b4, (0, 6))
    (t4,) = _mid_tc([a3a, a3b], [t3a, t3b], dinv, b3.reshape(1, 64), W4p,
                    64, 16, 1)
    a4 = _make_agg(16, True)(t4, packed)
    return _final_tc(a4, t4, dinv, b4p.reshape(1, 16), batch.reshape(N, 1))


# speedup vs baseline: 34.6535x; 1.1443x over previous
import functools

import jax
import jax.numpy as jnp
from jax import lax
from jax.experimental import pallas as pl
from jax.experimental.pallas import tpu as pltpu
from jax.experimental.pallas import tpu_sc as plsc

N = 10000
E = 320000
G = 64

NW = 32
CH = 128
NCH = 80
EP = NW * NCH * CH
NP = 10240
RPT = NP // 16

@functools.cache
def _sc_mesh():
    return plsc.VectorSubcoreMesh(
        core_axis_name="c", subcore_axis_name="s", num_cores=2, num_subcores=16
    )


def _zero_stage(stage_v, f):
    def zloop(i, _):
        for k in range(f // 16):
            stage_v[i, pl.ds(k * 16, 16)] = jnp.zeros((16,), jnp.float32)
        return _

    lax.fori_loop(0, RPT, zloop, None)


def _unpack_edges(pk_v, src_v, dst_v):
    def uloop(i, _):
        for k in range(CH // 16):
            v = pk_v[i, pl.ds(k * 16, 16)]
            src_v[i, pl.ds(k * 16, 16)] = v & 0xFFFF
            dst_v[i, pl.ds(k * 16, 16)] = v >> 16
        return _

    lax.fori_loop(0, NCH, uloop, None)


@functools.cache
def _make_deg():
    @functools.partial(
        pl.kernel,
        out_type=jax.ShapeDtypeStruct((2, NP, 16), jnp.float32),
        mesh=_sc_mesh(),
        compiler_params=pltpu.CompilerParams(use_tc_tiling_on_sc=False),
        scratch_types=[
            pltpu.VMEM((NCH, CH), jnp.int32),
            pltpu.VMEM((NCH, CH), jnp.int32),
            pltpu.VMEM((CH, 16), jnp.float32),
            pltpu.VMEM((RPT, 16), jnp.float32),
            pltpu.VMEM_SHARED((NP, 16), jnp.float32),
        ],
    )
    def deg_kernel(pk_hbm, out_hbm, pk_v, idx_v, ones_v, stage_v, acc_sh):
        c = lax.axis_index("c")
        s = lax.axis_index("s")
        w = c * 16 + s

        _zero_stage(stage_v, 16)

        def oloop(i, _):
            ones_v[i, :] = jnp.ones((16,), jnp.float32)
            return _

        lax.fori_loop(0, CH, oloop, None)

        pltpu.sync_copy(stage_v, acc_sh.at[pl.ds(s * RPT, RPT)])
        pltpu.sync_copy(pk_hbm.at[w], pk_v)
        _unpack_edges(pk_v, idx_v, idx_v)
        plsc.subcore_barrier()

        def body(j, _):
            pltpu.sync_copy(ones_v, acc_sh.at[idx_v.at[j]], add=True)
            return _

        lax.fori_loop(0, NCH, body, None)
        plsc.subcore_barrier()
        pltpu.sync_copy(acc_sh.at[pl.ds(s * RPT, RPT)],
                        out_hbm.at[c, pl.ds(s * RPT, RPT)])

    return deg_kernel


@functools.cache
def _make_agg(f, in_spmem):
    nbuf = 4

    @functools.partial(
        pl.kernel,
        out_type=jax.ShapeDtypeStruct((2, NP, f), jnp.float32),
        mesh=_sc_mesh(),
        compiler_params=pltpu.CompilerParams(use_tc_tiling_on_sc=False),
        scratch_types=[
            pltpu.VMEM((NCH, CH), jnp.int32),
            pltpu.VMEM((NCH, CH), jnp.int32),
            pltpu.VMEM((NCH, CH), jnp.int32),
            [pltpu.VMEM((CH, f), jnp.float32) for _ in range(nbuf)],
            [pltpu.SemaphoreType.DMA for _ in range(nbuf)],
            pltpu.VMEM_SHARED((NP, f), jnp.float32),
            (pltpu.VMEM_SHARED((N, f), jnp.float32) if in_spmem
             else pltpu.VMEM((16,), jnp.float32)),
        ],
    )
    def agg(table_hbm, pk_hbm, out_hbm, pk_v, src_v, dst_v, rows,
            sems, acc_sh, table_sh):
        c = lax.axis_index("c")
        s = lax.axis_index("s")
        w = c * 16 + s

        if in_spmem:
            tpt = N // 16
            pltpu.sync_copy(table_hbm.at[pl.ds(s * tpt, tpt)],
                            table_sh.at[pl.ds(s * tpt, tpt)])
        table = table_sh if in_spmem else table_hbm

        def zloop(i, _):
            for k in range(f // 16):
                rows[0][i, pl.ds(k * 16, 16)] = jnp.zeros((16,), jnp.float32)
            return _

        lax.fori_loop(0, CH, zloop, None)
        for r in range(RPT // CH):
            pltpu.sync_copy(rows[0], acc_sh.at[pl.ds(s * RPT + r * CH, CH)])
        pltpu.sync_copy(pk_hbm.at[w], pk_v)
        _unpack_edges(pk_v, src_v, dst_v)
        plsc.subcore_barrier()

        def body(jj, _):
            j = jj * nbuf
            cps = [
                pltpu.async_copy(table.at[src_v.at[j + b]], rows[b], sems[b])
                for b in range(nbuf)
            ]
            for b in range(nbuf):
                cps[b].wait()
                pltpu.sync_copy(rows[b], acc_sh.at[dst_v.at[j + b]], add=True)
            return _

        lax.fori_loop(0, NCH // nbuf, body, None)
        plsc.subcore_barrier()
        pltpu.sync_copy(acc_sh.at[pl.ds(s * RPT, RPT)],
                        out_hbm.at[c, pl.ds(s * RPT, RPT)])

    return agg


_BLK = 1000
_NBLK = N // _BLK


def _first_tc(x, W1, degp):
    def body(x_ref, w_ref, d0_ref, d1_ref, t_ref, dinv_ref):
        deg = d0_ref[0][:, :1] + d1_ref[0][:, :1] + 1.0
        dinv = lax.rsqrt(deg)
        xw = jnp.dot(x_ref[...], w_ref[...], preferred_element_type=jnp.float32)
        t_ref[...] = xw * dinv
        dinv_ref[...] = dinv

    return pl.pallas_call(
        body,
        grid=(_NBLK,),
        in_specs=[
            pl.BlockSpec((_BLK, 128), lambda i: (i, 0)),
            pl.BlockSpec((128, 16), lambda i: (0, 0)),
            pl.BlockSpec((1, _BLK, 16), lambda i: (0, i, 0)),
            pl.BlockSpec((1, _BLK, 16), lambda i: (1, i, 0)),
        ],
        out_specs=[
            pl.BlockSpec((_BLK, 16), lambda i: (i, 0)),
            pl.BlockSpec((_BLK, 1), lambda i: (i, 0)),
        ],
        out_shape=[
            jax.ShapeDtypeStruct((N, 16), jnp.float32),
            jax.ShapeDtypeStruct((N, 1), jnp.float32),
        ],
    )(x, W1, degp, degp)


def _mid_tc(accs, ts, dinv, b, Wn, f_in, f_out, nout):
    nin = len(ts)
    fh = f_in // nin
    fo = f_out // nout

    def body(*refs):
        a_refs = refs[: 2 * nin]
        t_refs = refs[2 * nin: 3 * nin]
        dinv_ref, b_ref, w_ref = refs[3 * nin: 3 * nin + 3]
        o_refs = refs[3 * nin + 3:]
        dinv = dinv_ref[...]
        parts = []
        for k in range(nin):
            hk = (dinv * (a_refs[2 * k][0] + a_refs[2 * k + 1][0] + t_refs[k][...])
                  + b_ref[:, k * fh:(k + 1) * fh])
            parts.append(hk)
        h = jnp.maximum(parts[0] if nin == 1 else jnp.concatenate(parts, axis=1), 0.0)
        for m in range(nout):
            o_refs[m][...] = jnp.dot(
                h, w_ref[:, m * fo:(m + 1) * fo],
                preferred_element_type=jnp.float32) * dinv

    in_specs = []
    args = []
    for k in range(nin):
        in_specs += [
            pl.BlockSpec((1, _BLK, fh), lambda i: (0, i, 0)),
            pl.BlockSpec((1, _BLK, fh), lambda i: (1, i, 0)),
        ]
        args += [accs[k], accs[k]]
    for k in range(nin):
        in_specs.append(pl.BlockSpec((_BLK, fh), lambda i: (i, 0)))
        args.append(ts[k])
    in_specs += [
        pl.BlockSpec((_BLK, 1), lambda i: (i, 0)),
        pl.BlockSpec((1, f_in), lambda i: (0, 0)),
        pl.BlockSpec((f_in, f_out), lambda i: (0, 0)),
    ]
    args += [dinv, b, Wn]
    out = pl.pallas_call(
        body,
        grid=(_NBLK,),
        in_specs=in_specs,
        out_specs=[pl.BlockSpec((_BLK, fo), lambda i: (i, 0))] * nout,
        out_shape=[jax.ShapeDtypeStruct((N, fo), jnp.float32)] * nout,
    )(*args)
    return out


def _final_tc(acc, t, dinv, b, batch2d):
    def body(a0_ref, a1_ref, t_ref, dinv_ref, b_ref, bt_ref, o_ref, pool_ref):
        i = pl.program_id(0)

        @pl.when(i == 0)
        def _():
            pool_ref[...] = jnp.full((16, G), -jnp.inf, jnp.float32)

        h = dinv_ref[...] * (a0_ref[0] + a1_ref[0] + t_ref[...]) + b_ref[...]
        mask = bt_ref[...] == lax.broadcasted_iota(jnp.int32, (1, G), 1)
        for f in range(16):
            cur = jnp.max(jnp.where(mask, h[:, f:f + 1], -jnp.inf), axis=0)
            pool_ref[f, :] = jnp.maximum(pool_ref[f, :], cur)

        @pl.when(i == _NBLK - 1)
        def _():
            p = pool_ref[...].T
            valid = lax.broadcasted_iota(jnp.int32, (G, 16), 1) < 10
            pm = jnp.where(valid, p, -jnp.inf)
            mx = jnp.max(pm, axis=1, keepdims=True)
            lse = jnp.log(jnp.sum(jnp.where(valid, jnp.exp(pm - mx), 0.0),
                                  axis=1, keepdims=True)) + mx
            o_ref[...] = (p - lse)[:, :10]

    return pl.pallas_call(
        body,
        grid=(_NBLK,),
        in_specs=[
            pl.BlockSpec((1, _BLK, 16), lambda i: (0, i, 0)),
            pl.BlockSpec((1, _BLK, 16), lambda i: (1, i, 0)),
            pl.BlockSpec((_BLK, 16), lambda i: (i, 0)),
            pl.BlockSpec((_BLK, 1), lambda i: (i, 0)),
            pl.BlockSpec((1, 16), lambda i: (0, 0)),
            pl.BlockSpec((_BLK, 1), lambda i: (i, 0)),
        ],
        out_specs=pl.BlockSpec((G, 10), lambda i: (0, 0)),
        out_shape=jax.ShapeDtypeStruct((G, 10), jnp.float32),
        scratch_shapes=[pltpu.VMEM((16, G), jnp.float32)],
    )(acc, acc, t, dinv, b, batch2d)


def kernel(x, edge_index, batch, W1, b1, W2, b2, W3, b3, W4, b4):
    src = edge_index[0]
    dst = edge_index[1]
    pad = EP - E
    packed = jnp.concatenate(
        [(dst << 16) | src, jnp.full((pad,), N << 16, jnp.int32)]
    ).reshape(NW, NCH, CH)

    degp = _make_deg()(packed)

    t1, dinv = _first_tc(x, W1, degp)
    a1 = _make_agg(16, True)(t1, packed)
    (t2,) = _mid_tc([a1], [t1], dinv, b1.reshape(1, 16), W2, 16, 32, 1)
    a2 = _make_agg(32, True)(t2, packed)
    t3a, t3b = _mid_tc([a2], [t2], dinv, b2.reshape(1, 32), W3, 32, 64, 2)
    a3a = _make_agg(32, True)(t3a, packed)
    a3b = _make_agg(32, True)(t3b, packed)
    W4p = jnp.pad(W4, ((0, 0), (0, 6)))
    b4p = jnp.pad(b4, (0, 6))
    (t4,) = _mid_tc([a3a, a3b], [t3a, t3b], dinv, b3.reshape(1, 64), W4p,
                    64, 16, 1)
    a4 = _make_agg(16, True)(t4, packed)
    return _final_tc(a4, t4, dinv, b4p.reshape(1, 16), batch.reshape(N, 1))
